# baseline (speedup 1.0000x reference)
import functools

import jax
import jax.numpy as jnp
from jax import lax
from jax.experimental import pallas as pl
from jax.experimental.pallas import tpu as pltpu
from jax.experimental.pallas import tpu_sc as plsc

B = 4096
DIM = 64
TOK = 8
NC, NS, L = 2, 16, 16
NW = NC * NS
TW = B // NW
G = TW // L
MARGIN = 1.0


def _rsqrt(x):
    i = plsc.bitcast(x, jnp.int32)
    y = plsc.bitcast(jnp.int32(0x5F3759DF) - (i >> 1), jnp.float32)
    for _ in range(3):
        y = y * (1.5 - 0.5 * x * y * y)
    return y


def _sc_body(eh_f, et_f, rr_f, th_f, tt_f, tr_f, wemb2,
             loss_o, pd_o, nd_o,
             ehb, etb, rrb, thb, ttb, trb,
             wring, widx, parb, tm, dv, lv,
             sem_w):
    wid = lax.axis_index("s") * NC + lax.axis_index("c")
    iota = lax.broadcasted_iota(jnp.int32, (L,), 0)
    zf = jnp.zeros((L,), jnp.float32)
    tok_bufs = (thb, trb, ttb)

    for set_id in range(2):
        tbase = set_id * B + wid * TW
        pltpu.sync_copy(eh_f.at[pl.ds(pl.multiple_of(tbase * DIM, 8), TW * DIM)], ehb)
        pltpu.sync_copy(et_f.at[pl.ds(pl.multiple_of(tbase * DIM, 8), TW * DIM)], etb)
        pltpu.sync_copy(rr_f.at[pl.ds(pl.multiple_of(tbase * DIM, 8), TW * DIM)], rrb)
        pltpu.sync_copy(th_f.at[pl.ds(pl.multiple_of(tbase * TOK, 8), TW * TOK)], thb)
        pltpu.sync_copy(tt_f.at[pl.ds(pl.multiple_of(tbase * TOK, 8), TW * TOK)], ttb)
        pltpu.sync_copy(tr_f.at[pl.ds(pl.multiple_of(tbase * TOK, 8), TW * TOK)], trb)

        def one_group(g, _):
            for t, tkb in enumerate(tok_bufs):
                for j in range(TOK):
                    tv = plsc.load_gather(tkb, [(g * L + iota) * TOK + j])
                    widx[pl.ds((t * TOK + j) * L, L)] = tv >> 1
                    parb[pl.ds((t * TOK + j) * L, L)] = (tv & 1) * DIM
            copies = [
                pltpu.async_copy(wemb2.at[widx.at[pl.ds(t * 128, 128)]],
                                 wring.at[pl.ds(t * 128, 128)], sem_w)
                for t in range(3)
            ]
            for cp in copies:
                cp.wait()

            def tm_zero(i, _):
                tm[pl.ds(i * L, L)] = zf
                return 0

            lax.fori_loop(0, 3 * DIM, tm_zero, 0)

            def tm_acc(k, _):
                t = k // TOK
                parv = parb[pl.ds(k * L, L)]
                rowv = k * L + iota
                for d in range(DIM):
                    x = plsc.load_gather(wring, [rowv, parv + d])
                    plsc.addupdate(tm.at[pl.ds((t * DIM + d) * L, L)], x)
                return 0

            lax.fori_loop(0, 3 * TOK, tm_acc, 0)

            rows0 = (g * L + iota) * DIM

            def p1(d, carry):
                sh, st = carry
                a = plsc.load_gather(ehb, [rows0 + d])
                b = plsc.load_gather(etb, [rows0 + d])
                return sh + a * a, st + b * b

            sh, st = lax.fori_loop(0, DIM, p1, (zf, zf))
            inv_h = _rsqrt(sh) * 0.125
            inv_t = _rsqrt(st) * 0.125

            def p2(d, acc):
                eh = plsc.load_gather(ehb, [rows0 + d])
                et = plsc.load_gather(etb, [rows0 + d])
                rr = plsc.load_gather(rrb, [rows0 + d])
                th = tm[pl.ds((0 * DIM + d) * L, L)]
                tr = tm[pl.ds((1 * DIM + d) * L, L)]
                tt = tm[pl.ds((2 * DIM + d) * L, L)]
                v = eh * inv_h * th + rr * (tr * 0.125) - et * inv_t * tt
                return acc + jnp.abs(v)

            dist = lax.fori_loop(0, DIM, p2, zf)
            dv[pl.ds(set_id * TW + g * L, L)] = dist
            return 0

        lax.fori_loop(0, G, one_group, 0)

    for k in range(TW // L):
        s = pl.ds(k * L, L)
        lv[s] = jnp.maximum(dv[s] - dv[pl.ds(TW + k * L, L)] + MARGIN, 0.0)

    obase = pl.multiple_of(wid * TW, 8)
    pltpu.sync_copy(lv, loss_o.at[pl.ds(obase, TW)])
    pltpu.sync_copy(dv.at[pl.ds(0, TW)], pd_o.at[pl.ds(obase, TW)])
    pltpu.sync_copy(dv.at[pl.ds(TW, TW)], nd_o.at[pl.ds(obase, TW)])


@jax.jit
def _sc_call(eh_f, et_f, rr_f, th_f, tt_f, tr_f, wemb2):
    mesh = plsc.VectorSubcoreMesh(core_axis_name="c", subcore_axis_name="s")
    f32 = jnp.float32
    i32 = jnp.int32
    run = functools.partial(
        pl.kernel,
        out_type=[jax.ShapeDtypeStruct((B,), f32)] * 3,
        mesh=mesh,
        compiler_params=pltpu.CompilerParams(
            use_tc_tiling_on_sc=True, needs_layout_passes=False),
        scratch_types=[
            pltpu.VMEM((TW * DIM,), f32),
            pltpu.VMEM((TW * DIM,), f32),
            pltpu.VMEM((TW * DIM,), f32),
            pltpu.VMEM((TW * TOK,), i32),
            pltpu.VMEM((TW * TOK,), i32),
            pltpu.VMEM((TW * TOK,), i32),
            pltpu.VMEM((3 * TOK * L, 128), f32),
            pltpu.VMEM((3 * TOK * L,), i32),
            pltpu.VMEM((3 * TOK * L,), i32),
            pltpu.VMEM((3 * DIM * L,), f32),
            pltpu.VMEM((2 * TW,), f32),
            pltpu.VMEM((TW,), f32),
            pltpu.SemaphoreType.DMA,
        ],
    )(_sc_body)
    return run(eh_f, et_f, rr_f, th_f, tt_f, tr_f, wemb2)


def kernel(positive_triplets, negative_triplets, entities_emb, relations_emb,
           word_emb, entity_token_ids, relation_token_ids):
    hs = jnp.concatenate([positive_triplets[:, 0], negative_triplets[:, 0]])
    rs = jnp.concatenate([positive_triplets[:, 1], negative_triplets[:, 1]])
    ts = jnp.concatenate([positive_triplets[:, 2], negative_triplets[:, 2]])
    eh_f = jnp.take(entities_emb.T, hs, axis=1).T.reshape(-1)
    et_f = jnp.take(entities_emb.T, ts, axis=1).T.reshape(-1)
    rr_f = jnp.take(relations_emb.T, rs, axis=1).T.reshape(-1)
    th_f = entity_token_ids[hs].reshape(-1)
    tt_f = entity_token_ids[ts].reshape(-1)
    tr_f = relation_token_ids[rs].reshape(-1)
    wemb2 = word_emb.reshape(16000, 128)
    loss, pd, nd = _sc_call(eh_f, et_f, rr_f, th_f, tt_f, tr_f, wemb2)
    return (loss, pd, nd)


# speedup vs baseline: 1.1725x; 1.1725x over previous
import functools

import jax
import jax.numpy as jnp
from jax import lax
from jax.experimental import pallas as pl
from jax.experimental.pallas import tpu as pltpu
from jax.experimental.pallas import tpu_sc as plsc

B = 4096
DIM = 64
TOK = 8
NC, NS, L = 2, 16, 16
NW = NC * NS
TW = B // NW
G = TW // L
MARGIN = 1.0


def _rsqrt(x):
    i = plsc.bitcast(x, jnp.int32)
    y = plsc.bitcast(jnp.int32(0x5F3759DF) - (i >> 1), jnp.float32)
    for _ in range(3):
        y = y * (1.5 - 0.5 * x * y * y)
    return y


def _sc_body(eh_f, et_f, rr_f, th_f, tt_f, tr_f, wemb,
             loss_o, pd_o, nd_o,
             ehb, etb, rrb, thb, ttb, trb,
             wrA, wrB, wiA, wiB, pbuf, dv, lv,
             semA, semB):
    wid = lax.axis_index("s") * NC + lax.axis_index("c")
    iota = lax.broadcasted_iota(jnp.int32, (L,), 0)
    zf = jnp.zeros((L,), jnp.float32)
    tok_bufs = (thb, trb, ttb)

    def build_fire(g, wi, wr, sem):
        for t, tkb in enumerate(tok_bufs):
            for j in range(TOK):
                tv = plsc.load_gather(tkb, [(g * L + iota) * TOK + j])
                wi[pl.ds((t * TOK + j) * L, L)] = tv
        for t in range(3):
            pltpu.async_copy(wemb.at[wi.at[pl.ds(t * 128, 128)]],
                             wr.at[pl.ds(t * 128, 128)], sem)

    def drain(wi, wr, sem):
        for t in range(3):
            pltpu.make_async_copy(wemb.at[wi.at[pl.ds(t * 128, 128)]],
                                  wr.at[pl.ds(t * 128, 128)], sem).wait()

    def compute(set_id, g, wr):
        rows0 = (g * L + iota) * DIM

        def main(d, carry):
            sh, st = carry
            sums = []
            for t in range(3):
                s = plsc.load_gather(wr, [(t * TOK) * L + iota, lax.broadcast(d, (L,))])
                for j in range(1, TOK):
                    s = s + plsc.load_gather(
                        wr, [(t * TOK + j) * L + iota, lax.broadcast(d, (L,))])
                sums.append(s)
            eh = plsc.load_gather(ehb, [rows0 + d])
            et = plsc.load_gather(etb, [rows0 + d])
            rr = plsc.load_gather(rrb, [rows0 + d])
            pbuf[pl.ds(0 * DIM * L + d * L, L)] = eh * sums[0]
            pbuf[pl.ds(1 * DIM * L + d * L, L)] = rr * sums[1]
            pbuf[pl.ds(2 * DIM * L + d * L, L)] = et * sums[2]
            return sh + eh * eh, st + et * et

        sh, st = lax.fori_loop(0, DIM, main, (zf, zf))
        inv_h = _rsqrt(sh) * 0.125
        inv_t = _rsqrt(st) * 0.125

        def p2(d, acc):
            a = pbuf[pl.ds(0 * DIM * L + d * L, L)]
            b = pbuf[pl.ds(1 * DIM * L + d * L, L)]
            c = pbuf[pl.ds(2 * DIM * L + d * L, L)]
            return acc + jnp.abs(a * inv_h + b * 0.125 - c * inv_t)

        dist = lax.fori_loop(0, DIM, p2, zf)
        dv[pl.ds(set_id * TW + g * L, L)] = dist

    for set_id in range(2):
        tbase = set_id * B + wid * TW
        pltpu.sync_copy(eh_f.at[pl.ds(pl.multiple_of(tbase * DIM, 8), TW * DIM)], ehb)
        pltpu.sync_copy(et_f.at[pl.ds(pl.multiple_of(tbase * DIM, 8), TW * DIM)], etb)
        pltpu.sync_copy(rr_f.at[pl.ds(pl.multiple_of(tbase * DIM, 8), TW * DIM)], rrb)
        pltpu.sync_copy(th_f.at[pl.ds(pl.multiple_of(tbase * TOK, 8), TW * TOK)], thb)
        pltpu.sync_copy(tt_f.at[pl.ds(pl.multiple_of(tbase * TOK, 8), TW * TOK)], ttb)
        pltpu.sync_copy(tr_f.at[pl.ds(pl.multiple_of(tbase * TOK, 8), TW * TOK)], trb)

        build_fire(0, wiA, wrA, semA)

        def pair(gp, _):
            ga = gp * 2
            build_fire(ga + 1, wiB, wrB, semB)
            drain(wiA, wrA, semA)
            compute(set_id, ga, wrA)

            @pl.when(gp < G // 2 - 1)
            def _():
                build_fire(ga + 2, wiA, wrA, semA)

            drain(wiB, wrB, semB)
            compute(set_id, ga + 1, wrB)
            return 0

        lax.fori_loop(0, G // 2, pair, 0)

    for k in range(TW // L):
        s = pl.ds(k * L, L)
        lv[s] = jnp.maximum(dv[s] - dv[pl.ds(TW + k * L, L)] + MARGIN, 0.0)

    obase = pl.multiple_of(wid * TW, 8)
    pltpu.sync_copy(lv, loss_o.at[pl.ds(obase, TW)])
    pltpu.sync_copy(dv.at[pl.ds(0, TW)], pd_o.at[pl.ds(obase, TW)])
    pltpu.sync_copy(dv.at[pl.ds(TW, TW)], nd_o.at[pl.ds(obase, TW)])


@jax.jit
def _sc_call(eh_f, et_f, rr_f, th_f, tt_f, tr_f, wemb):
    mesh = plsc.VectorSubcoreMesh(core_axis_name="c", subcore_axis_name="s")
    f32 = jnp.float32
    i32 = jnp.int32
    run = functools.partial(
        pl.kernel,
        out_type=[jax.ShapeDtypeStruct((B,), f32)] * 3,
        mesh=mesh,
        compiler_params=pltpu.CompilerParams(
            use_tc_tiling_on_sc=False, needs_layout_passes=False),
        scratch_types=[
            pltpu.VMEM((TW * DIM,), f32),
            pltpu.VMEM((TW * DIM,), f32),
            pltpu.VMEM((TW * DIM,), f32),
            pltpu.VMEM((TW * TOK,), i32),
            pltpu.VMEM((TW * TOK,), i32),
            pltpu.VMEM((TW * TOK,), i32),
            pltpu.VMEM((3 * TOK * L, DIM), f32),
            pltpu.VMEM((3 * TOK * L, DIM), f32),
            pltpu.VMEM((3 * TOK * L,), i32),
            pltpu.VMEM((3 * TOK * L,), i32),
            pltpu.VMEM((3 * DIM * L,), f32),
            pltpu.VMEM((2 * TW,), f32),
            pltpu.VMEM((TW,), f32),
            pltpu.SemaphoreType.DMA,
            pltpu.SemaphoreType.DMA,
        ],
    )(_sc_body)
    return run(eh_f, et_f, rr_f, th_f, tt_f, tr_f, wemb)


def kernel(positive_triplets, negative_triplets, entities_emb, relations_emb,
           word_emb, entity_token_ids, relation_token_ids):
    hs = jnp.concatenate([positive_triplets[:, 0], negative_triplets[:, 0]])
    rs = jnp.concatenate([positive_triplets[:, 1], negative_triplets[:, 1]])
    ts = jnp.concatenate([positive_triplets[:, 2], negative_triplets[:, 2]])
    eh_f = entities_emb[hs].reshape(-1)
    et_f = entities_emb[ts].reshape(-1)
    rr_f = relations_emb[rs].reshape(-1)
    th_f = entity_token_ids[hs].reshape(-1)
    tt_f = entity_token_ids[ts].reshape(-1)
    tr_f = relation_token_ids[rs].reshape(-1)
    loss, pd, nd = _sc_call(eh_f, et_f, rr_f, th_f, tt_f, tr_f, word_emb)
    return (loss, pd, nd)


# speedup vs baseline: 1.1829x; 1.0088x over previous
import functools

import jax
import jax.numpy as jnp
from jax import lax
from jax.experimental import pallas as pl
from jax.experimental.pallas import tpu as pltpu
from jax.experimental.pallas import tpu_sc as plsc

B = 4096
DIM = 64
TOK = 8
NC, NS, L = 2, 16, 16
NW = NC * NS
TW = B // NW
G = TW // L
MARGIN = 1.0


def _rsqrt(x):
    i = plsc.bitcast(x, jnp.int32)
    y = plsc.bitcast(jnp.int32(0x5F3759DF) - (i >> 1), jnp.float32)
    for _ in range(3):
        y = y * (1.5 - 0.5 * x * y * y)
    return y


def _sc_body(eh_f, et_f, rr_f, th_f, tt_f, tr_f, wemb,
             loss_o, pd_o, nd_o,
             ehb, etb, rrb, thb, ttb, trb,
             wrA, wrB, wiA, wiB, pbuf, dv, lv,
             semA, semB):
    wid = lax.axis_index("s") * NC + lax.axis_index("c")
    iota = lax.broadcasted_iota(jnp.int32, (L,), 0)
    zf = jnp.zeros((L,), jnp.float32)
    tok_bufs = (thb, trb, ttb)

    def build_fire(g, wi, wr, sem):
        for t, tkb in enumerate(tok_bufs):
            for j in range(TOK):
                tv = plsc.load_gather(tkb, [(g * L + iota) * TOK + j])
                wi[pl.ds((t * TOK + j) * L, L)] = tv
        for t in range(3):
            pltpu.async_copy(wemb.at[wi.at[pl.ds(t * 128, 128)]],
                             wr.at[pl.ds(t * 128, 128)], sem)

    def drain(wi, wr, sem):
        for t in range(3):
            pltpu.make_async_copy(wemb.at[wi.at[pl.ds(t * 128, 128)]],
                                  wr.at[pl.ds(t * 128, 128)], sem).wait()

    def compute(set_id, g, wr):
        rows0 = (g * L + iota) * DIM

        def main(dd, carry):
            sh, st = carry
            for u in range(2):
                d = dd * 2 + u
                sums = []
                for t in range(3):
                    s = plsc.load_gather(wr, [(t * TOK) * L + iota, lax.broadcast(d, (L,))])
                    for j in range(1, TOK):
                        s = s + plsc.load_gather(
                            wr, [(t * TOK + j) * L + iota, lax.broadcast(d, (L,))])
                    sums.append(s)
                eh = plsc.load_gather(ehb, [rows0 + d])
                et = plsc.load_gather(etb, [rows0 + d])
                rr = plsc.load_gather(rrb, [rows0 + d])
                pbuf[pl.ds(0 * DIM * L + d * L, L)] = eh * sums[0]
                pbuf[pl.ds(1 * DIM * L + d * L, L)] = rr * sums[1]
                pbuf[pl.ds(2 * DIM * L + d * L, L)] = et * sums[2]
                sh = sh + eh * eh
                st = st + et * et
            return sh, st

        sh, st = lax.fori_loop(0, DIM // 2, main, (zf, zf))
        inv_h = _rsqrt(sh) * 0.125
        inv_t = _rsqrt(st) * 0.125

        def p2(d, acc):
            a = pbuf[pl.ds(0 * DIM * L + d * L, L)]
            b = pbuf[pl.ds(1 * DIM * L + d * L, L)]
            c = pbuf[pl.ds(2 * DIM * L + d * L, L)]
            return acc + jnp.abs(a * inv_h + b * 0.125 - c * inv_t)

        dist = lax.fori_loop(0, DIM, p2, zf)
        dv[pl.ds(set_id * TW + g * L, L)] = dist

    for set_id in range(2):
        tbase = set_id * B + wid * TW
        stage = [
            pltpu.async_copy(eh_f.at[pl.ds(pl.multiple_of(tbase * DIM, 8), TW * DIM)], ehb, semA),
            pltpu.async_copy(et_f.at[pl.ds(pl.multiple_of(tbase * DIM, 8), TW * DIM)], etb, semA),
            pltpu.async_copy(rr_f.at[pl.ds(pl.multiple_of(tbase * DIM, 8), TW * DIM)], rrb, semA),
            pltpu.async_copy(th_f.at[pl.ds(pl.multiple_of(tbase * TOK, 8), TW * TOK)], thb, semA),
            pltpu.async_copy(tt_f.at[pl.ds(pl.multiple_of(tbase * TOK, 8), TW * TOK)], ttb, semA),
            pltpu.async_copy(tr_f.at[pl.ds(pl.multiple_of(tbase * TOK, 8), TW * TOK)], trb, semA),
        ]
        for cp in stage:
            cp.wait()

        build_fire(0, wiA, wrA, semA)

        def pair(gp, _):
            ga = gp * 2
            build_fire(ga + 1, wiB, wrB, semB)
            drain(wiA, wrA, semA)
            compute(set_id, ga, wrA)

            @pl.when(gp < G // 2 - 1)
            def _():
                build_fire(ga + 2, wiA, wrA, semA)

            drain(wiB, wrB, semB)
            compute(set_id, ga + 1, wrB)
            return 0

        lax.fori_loop(0, G // 2, pair, 0)

    for k in range(TW // L):
        s = pl.ds(k * L, L)
        lv[s] = jnp.maximum(dv[s] - dv[pl.ds(TW + k * L, L)] + MARGIN, 0.0)

    obase = pl.multiple_of(wid * TW, 8)
    pltpu.sync_copy(lv, loss_o.at[pl.ds(obase, TW)])
    pltpu.sync_copy(dv.at[pl.ds(0, TW)], pd_o.at[pl.ds(obase, TW)])
    pltpu.sync_copy(dv.at[pl.ds(TW, TW)], nd_o.at[pl.ds(obase, TW)])


@jax.jit
def _sc_call(eh_f, et_f, rr_f, th_f, tt_f, tr_f, wemb):
    mesh = plsc.VectorSubcoreMesh(core_axis_name="c", subcore_axis_name="s")
    f32 = jnp.float32
    i32 = jnp.int32
    run = functools.partial(
        pl.kernel,
        out_type=[jax.ShapeDtypeStruct((B,), f32)] * 3,
        mesh=mesh,
        compiler_params=pltpu.CompilerParams(
            use_tc_tiling_on_sc=False, needs_layout_passes=False),
        scratch_types=[
            pltpu.VMEM((TW * DIM,), f32),
            pltpu.VMEM((TW * DIM,), f32),
            pltpu.VMEM((TW * DIM,), f32),
            pltpu.VMEM((TW * TOK,), i32),
            pltpu.VMEM((TW * TOK,), i32),
            pltpu.VMEM((TW * TOK,), i32),
            pltpu.VMEM((3 * TOK * L, DIM), f32),
            pltpu.VMEM((3 * TOK * L, DIM), f32),
            pltpu.VMEM((3 * TOK * L,), i32),
            pltpu.VMEM((3 * TOK * L,), i32),
            pltpu.VMEM((3 * DIM * L,), f32),
            pltpu.VMEM((2 * TW,), f32),
            pltpu.VMEM((TW,), f32),
            pltpu.SemaphoreType.DMA,
            pltpu.SemaphoreType.DMA,
        ],
    )(_sc_body)
    return run(eh_f, et_f, rr_f, th_f, tt_f, tr_f, wemb)


def kernel(positive_triplets, negative_triplets, entities_emb, relations_emb,
           word_emb, entity_token_ids, relation_token_ids):
    hs = jnp.concatenate([positive_triplets[:, 0], negative_triplets[:, 0]])
    rs = jnp.concatenate([positive_triplets[:, 1], negative_triplets[:, 1]])
    ts = jnp.concatenate([positive_triplets[:, 2], negative_triplets[:, 2]])
    eh_f = entities_emb[hs].reshape(-1)
    et_f = entities_emb[ts].reshape(-1)
    rr_f = relations_emb[rs].reshape(-1)
    th_f = entity_token_ids[hs].reshape(-1)
    tt_f = entity_token_ids[ts].reshape(-1)
    tr_f = relation_token_ids[rs].reshape(-1)
    loss, pd, nd = _sc_call(eh_f, et_f, rr_f, th_f, tt_f, tr_f, word_emb)
    return (loss, pd, nd)


# speedup vs baseline: 1.1847x; 1.0015x over previous
import functools

import jax
import jax.numpy as jnp
from jax import lax
from jax.experimental import pallas as pl
from jax.experimental.pallas import tpu as pltpu
from jax.experimental.pallas import tpu_sc as plsc

B = 4096
DIM = 64
TOK = 8
NC, NS, L = 2, 16, 16
NW = NC * NS
TW = B // NW
G = TW // L
MARGIN = 1.0


def _rsqrt(x):
    i = plsc.bitcast(x, jnp.int32)
    y = plsc.bitcast(jnp.int32(0x5F3759DF) - (i >> 1), jnp.float32)
    for _ in range(3):
        y = y * (1.5 - 0.5 * x * y * y)
    return y


def _sc_body(eh_f, et_f, rr_f, th_f, tt_f, tr_f, wemb,
             loss_o, pd_o, nd_o,
             ehb, etb, rrb, thb, ttb, trb,
             wrA, wrB, wiA, wiB, pbuf, dv, lv,
             semA, semB):
    wid = lax.axis_index("s") * NC + lax.axis_index("c")
    iota = lax.broadcasted_iota(jnp.int32, (L,), 0)
    zf = jnp.zeros((L,), jnp.float32)
    tok_bufs = (thb, trb, ttb)

    def build_fire(g, wi, wr, sem):
        for t, tkb in enumerate(tok_bufs):
            for j in range(TOK):
                tv = plsc.load_gather(tkb, [(g * L + iota) * TOK + j])
                wi[pl.ds((t * TOK + j) * L, L)] = tv
        for t in range(3):
            pltpu.async_copy(wemb.at[wi.at[pl.ds(t * 128, 128)]],
                             wr.at[pl.ds(t * 128, 128)], sem)

    def drain(wi, wr, sem):
        for t in range(3):
            pltpu.make_async_copy(wemb.at[wi.at[pl.ds(t * 128, 128)]],
                                  wr.at[pl.ds(t * 128, 128)], sem).wait()

    def compute(set_id, g, wr):
        rows0 = (g * L + iota) * DIM

        def main(dd, carry):
            sh, st = carry
            for u in range(2):
                d = dd * 2 + u
                sums = []
                for t in range(3):
                    s = plsc.load_gather(wr, [(t * TOK) * L + iota, lax.broadcast(d, (L,))])
                    for j in range(1, TOK):
                        s = s + plsc.load_gather(
                            wr, [(t * TOK + j) * L + iota, lax.broadcast(d, (L,))])
                    sums.append(s)
                eh = plsc.load_gather(ehb, [rows0 + d])
                et = plsc.load_gather(etb, [rows0 + d])
                rr = plsc.load_gather(rrb, [rows0 + d])
                pbuf[pl.ds(0 * DIM * L + d * L, L)] = eh * sums[0]
                pbuf[pl.ds(1 * DIM * L + d * L, L)] = rr * sums[1]
                pbuf[pl.ds(2 * DIM * L + d * L, L)] = et * sums[2]
                sh = sh + eh * eh
                st = st + et * et
            return sh, st

        sh, st = lax.fori_loop(0, DIM // 2, main, (zf, zf))
        inv_h = _rsqrt(sh) * 0.125
        inv_t = _rsqrt(st) * 0.125

        def p2(d, acc):
            a = pbuf[pl.ds(0 * DIM * L + d * L, L)]
            b = pbuf[pl.ds(1 * DIM * L + d * L, L)]
            c = pbuf[pl.ds(2 * DIM * L + d * L, L)]
            return acc + jnp.abs(a * inv_h + b * 0.125 - c * inv_t)

        dist = lax.fori_loop(0, DIM, p2, zf)
        dv[pl.ds(set_id * TW + g * L, L)] = dist

    for set_id in range(2):
        tbase = set_id * B + wid * TW
        stage = [
            pltpu.async_copy(eh_f.at[pl.ds(pl.multiple_of(tbase * DIM, 8), TW * DIM)], ehb, semA),
            pltpu.async_copy(et_f.at[pl.ds(pl.multiple_of(tbase * DIM, 8), TW * DIM)], etb, semA),
            pltpu.async_copy(rr_f.at[pl.ds(pl.multiple_of(tbase * DIM, 8), TW * DIM)], rrb, semA),
            pltpu.async_copy(th_f.at[pl.ds(pl.multiple_of(tbase * TOK, 8), TW * TOK)], thb, semA),
            pltpu.async_copy(tt_f.at[pl.ds(pl.multiple_of(tbase * TOK, 8), TW * TOK)], ttb, semA),
            pltpu.async_copy(tr_f.at[pl.ds(pl.multiple_of(tbase * TOK, 8), TW * TOK)], trb, semA),
        ]
        for cp in stage:
            cp.wait()

        build_fire(0, wiA, wrA, semA)

        def pair(gp, _):
            ga = gp * 2
            build_fire(ga + 1, wiB, wrB, semB)
            drain(wiA, wrA, semA)
            compute(set_id, ga, wrA)

            @pl.when(gp < G // 2 - 1)
            def _():
                build_fire(ga + 2, wiA, wrA, semA)

            drain(wiB, wrB, semB)
            compute(set_id, ga + 1, wrB)
            return 0

        lax.fori_loop(0, G // 2, pair, 0)

    for k in range(TW // L):
        s = pl.ds(k * L, L)
        lv[s] = jnp.maximum(dv[s] - dv[pl.ds(TW + k * L, L)] + MARGIN, 0.0)

    obase = pl.multiple_of(wid * TW, 8)
    pltpu.sync_copy(lv, loss_o.at[pl.ds(obase, TW)])
    pltpu.sync_copy(dv.at[pl.ds(0, TW)], pd_o.at[pl.ds(obase, TW)])
    pltpu.sync_copy(dv.at[pl.ds(TW, TW)], nd_o.at[pl.ds(obase, TW)])


@jax.jit
def _sc_call(eh_f, et_f, rr_f, th_f, tt_f, tr_f, wemb):
    mesh = plsc.VectorSubcoreMesh(core_axis_name="c", subcore_axis_name="s")
    f32 = jnp.float32
    i32 = jnp.int32
    run = functools.partial(
        pl.kernel,
        out_type=[jax.ShapeDtypeStruct((B,), f32)] * 3,
        mesh=mesh,
        compiler_params=pltpu.CompilerParams(
            use_tc_tiling_on_sc=False, needs_layout_passes=False,
            disable_bounds_checks=True),
        scratch_types=[
            pltpu.VMEM((TW * DIM,), f32),
            pltpu.VMEM((TW * DIM,), f32),
            pltpu.VMEM((TW * DIM,), f32),
            pltpu.VMEM((TW * TOK,), i32),
            pltpu.VMEM((TW * TOK,), i32),
            pltpu.VMEM((TW * TOK,), i32),
            pltpu.VMEM((3 * TOK * L, DIM), f32),
            pltpu.VMEM((3 * TOK * L, DIM), f32),
            pltpu.VMEM((3 * TOK * L,), i32),
            pltpu.VMEM((3 * TOK * L,), i32),
            pltpu.VMEM((3 * DIM * L,), f32),
            pltpu.VMEM((2 * TW,), f32),
            pltpu.VMEM((TW,), f32),
            pltpu.SemaphoreType.DMA,
            pltpu.SemaphoreType.DMA,
        ],
    )(_sc_body)
    return run(eh_f, et_f, rr_f, th_f, tt_f, tr_f, wemb)


def kernel(positive_triplets, negative_triplets, entities_emb, relations_emb,
           word_emb, entity_token_ids, relation_token_ids):
    hs = jnp.concatenate([positive_triplets[:, 0], negative_triplets[:, 0]])
    rs = jnp.concatenate([positive_triplets[:, 1], negative_triplets[:, 1]])
    ts = jnp.concatenate([positive_triplets[:, 2], negative_triplets[:, 2]])
    eh_f = entities_emb[hs].reshape(-1)
    et_f = entities_emb[ts].reshape(-1)
    rr_f = relations_emb[rs].reshape(-1)
    th_f = entity_token_ids[hs].reshape(-1)
    tt_f = entity_token_ids[ts].reshape(-1)
    tr_f = relation_token_ids[rs].reshape(-1)
    loss, pd, nd = _sc_call(eh_f, et_f, rr_f, th_f, tt_f, tr_f, word_emb)
    return (loss, pd, nd)
